# Initial kernel scaffold; baseline (speedup 1.0000x reference)
import jax
import jax.numpy as jnp
from jax.experimental import pallas as pl


def kernel(anonymized_nodes, walk_masks, walk_times):
    raise NotImplementedError("write your pallas kernel here")


# speedup vs baseline: 229.8482x; 229.8482x over previous
import math

import jax
import jax.numpy as jnp
from jax import lax
from jax.experimental import pallas as pl
from jax.experimental.pallas import tpu as pltpu
from jax.experimental.pallas import tpu_sc as plsc

B = 16
W = 32
L = 20
M = W * L
LANES = 16
NCHUNK = M // LANES
SIGMA_DIST = 2.0
SIGMA_TIME = 5.0
INV_ST = 1.0 / SIGMA_TIME
DBAND = 6


def _sc_body(n_rm_hbm, t_rm_hbm, m_rm_hbm, nT_hbm, tT_hbm, mT_hbm, out_hbm,
             nrm, trm, mrm, nT, tT, mT, s1T, s2T, e1b, e2b, outbuf):
    cid = lax.axis_index("c")
    sid = lax.axis_index("s")
    wid = sid * 2 + cid
    b = wid // 2
    base_w1 = (wid % 2) * LANES

    pltpu.sync_copy(n_rm_hbm.at[b], nrm.at[pl.ds(0, M)])
    pltpu.sync_copy(t_rm_hbm.at[b], trm)
    pltpu.sync_copy(m_rm_hbm.at[b], mrm)
    pltpu.sync_copy(nT_hbm.at[b], nT)
    pltpu.sync_copy(tT_hbm.at[b], tT)
    pltpu.sync_copy(mT_hbm.at[b], mT)

    def pre_body(c, _):
        j0 = c * LANES
        tv = tT[pl.ds(j0, LANES)]
        bv = jnp.where(mT[pl.ds(j0, LANES)] != 0.0, 1.0, 0.0)
        s1T[pl.ds(j0, LANES)] = jnp.exp(tv * INV_ST) * bv
        s2T[pl.ds(j0, LANES)] = jnp.exp(tv * (-INV_ST)) * bv
        tv2 = trm[pl.ds(j0, LANES)]
        bv2 = jnp.where(mrm[pl.ds(j0, LANES)] != 0.0, 1.0, 0.0)
        e1b[pl.ds(j0, LANES)] = jnp.exp(tv2 * INV_ST) * bv2
        e2b[pl.ds(j0, LANES)] = jnp.exp(tv2 * (-INV_ST)) * bv2
        return 0

    lax.fori_loop(0, NCHUNK, pre_body, 0, unroll=False)

    kconst = [[math.exp(-((p1 - p2) ** 2) / (SIGMA_DIST ** 2))
               for p2 in range(L)] for p1 in range(L)]

    def w2_body(w2, _):
        r0 = w2 * L
        na = nrm[pl.ds(r0, LANES)]
        nb = nrm[pl.ds(r0 + 4, LANES)]
        f1a = e1b[pl.ds(r0, LANES)]
        f1b = e1b[pl.ds(r0 + 4, LANES)]
        f2a = e2b[pl.ds(r0, LANES)]
        f2b = e2b[pl.ds(r0 + 4, LANES)]
        n2 = [na[p] for p in range(LANES)] + [nb[p + 12] for p in range(L - LANES)]
        f1 = [f1a[p] for p in range(LANES)] + [f1b[p + 12] for p in range(L - LANES)]
        f2 = [f2a[p] for p in range(LANES)] + [f2b[p + 12] for p in range(L - LANES)]

        accs = [jnp.zeros((LANES,), jnp.float32) for _ in range(4)]
        k = 0
        for p1 in range(L):
            o1 = p1 * W + base_w1
            nTv = nT[pl.ds(o1, LANES)]
            s1v = s1T[pl.ds(o1, LANES)]
            s2v = s2T[pl.ds(o1, LANES)]
            for p2 in range(max(0, p1 - DBAND), min(L, p1 + DBAND + 1)):
                kc = kconst[p1][p2]
                x = s1v * (f2[p2] * kc)
                y = s2v * (f1[p2] * kc)
                tf = jnp.minimum(x, y)
                accs[k % 4] = accs[k % 4] + jnp.where(nTv == n2[p2], tf, 0.0)
                k += 1
        acc = (accs[0] + accs[1]) + (accs[2] + accs[3])

        acc = jnp.minimum(jnp.maximum(acc, -10.0), 10.0)
        e = jnp.exp(acc * 2.0)
        outbuf[pl.ds(w2 * LANES, LANES)] = 1.0 - 2.0 / (e + 1.0)
        return 0

    lax.fori_loop(0, W, w2_body, 0, unroll=False)

    pltpu.sync_copy(outbuf, out_hbm.at[wid])


@jax.jit
def _cooc(n_rm, t_rm, m_rm, nT, tT, mT):
    mesh = plsc.VectorSubcoreMesh(core_axis_name="c", subcore_axis_name="s")
    f = pl.kernel(
        _sc_body,
        out_type=jax.ShapeDtypeStruct((2 * B, LANES * W), jnp.float32),
        mesh=mesh,
        scratch_types=[
            pltpu.VMEM((M + LANES,), jnp.int32),
            pltpu.VMEM((M,), jnp.float32),
            pltpu.VMEM((M,), jnp.float32),
            pltpu.VMEM((M,), jnp.int32),
            pltpu.VMEM((M,), jnp.float32),
            pltpu.VMEM((M,), jnp.float32),
            pltpu.VMEM((M,), jnp.float32),
            pltpu.VMEM((M,), jnp.float32),
            pltpu.VMEM((M + LANES,), jnp.float32),
            pltpu.VMEM((M + LANES,), jnp.float32),
            pltpu.VMEM((LANES * W,), jnp.float32),
        ],
    )
    return f(n_rm, t_rm, m_rm, nT, tT, mT)


def kernel(anonymized_nodes, walk_masks, walk_times):
    nodes = anonymized_nodes.astype(jnp.int32)
    times = walk_times.astype(jnp.float32)
    masks = walk_masks.astype(jnp.float32)
    n_rm = nodes.reshape(B, M)
    t_rm = times.reshape(B, M)
    m_rm = masks.reshape(B, M)
    nT = nodes.transpose(0, 2, 1).reshape(B, M)
    tT = times.transpose(0, 2, 1).reshape(B, M)
    mT = masks.transpose(0, 2, 1).reshape(B, M)
    out32 = _cooc(n_rm, t_rm, m_rm, nT, tT, mT)
    return (out32.reshape(B, 2, W, LANES)
                 .transpose(0, 1, 3, 2)
                 .reshape(B, W, W))
